# Initial kernel scaffold; baseline (speedup 1.0000x reference)
import jax
import jax.numpy as jnp
from jax.experimental import pallas as pl


def kernel(laplacian_indices, laplacian_values, features, W1, b1, W2, b2):
    raise NotImplementedError("write your pallas kernel here")


# speedup vs baseline: 4.1698x; 4.1698x over previous
import functools

import jax
import jax.numpy as jnp
from jax import lax
from jax.experimental import pallas as pl
from jax.experimental.pallas import tpu as pltpu
from jax.experimental.pallas import tpu_sc as plsc

N = 10000
E = 320000
D = 128

NC = 2
NS = 16
NW = NC * NS
EPW = E // NW
CHUNK = 80
NCHUNK = EPW // CHUNK
ZR = 80
NZB = N // ZR


def _spmm_body(src_hbm, dst_hbm, vals_hbm, table_hbm, out_hbm,
               acc, srcv, dstv, valv, rows, zbuf, sem):
    cid = lax.axis_index("c")
    sid = lax.axis_index("s")
    wid = cid * NS + sid

    zero16 = jnp.zeros((16,), jnp.float32)

    def zb(i, c):
        for j in range(8):
            zbuf[i, pl.ds(j * 16, 16)] = zero16
        return c

    lax.fori_loop(0, ZR, zb, 0)

    for k in range((NZB + NS - 1) // NS):
        b = k * NS + sid

        @pl.when(b < NZB)
        def _():
            base = pl.multiple_of(b * ZR, 8)
            pltpu.sync_copy(zbuf, acc.at[pl.ds(base, ZR)])

    plsc.subcore_barrier()

    def chunk_body(ch, c):
        base = pl.multiple_of(wid * EPW + ch * CHUNK, 8)
        pltpu.sync_copy(src_hbm.at[pl.ds(base, CHUNK)], srcv)
        pltpu.sync_copy(dst_hbm.at[pl.ds(base, CHUNK)], dstv)
        pltpu.sync_copy(vals_hbm.at[pl.ds(base, CHUNK)], valv)
        pltpu.async_copy(table_hbm.at[srcv], rows, sem).wait()

        def scale(g, cc):
            vv = valv[pl.ds(g * 16, 16)]
            rbase = g * 16
            for r in range(16):
                v = vv[r]
                for j in range(8):
                    sl = pl.ds(j * 16, 16)
                    rows[rbase + r, sl] = rows[rbase + r, sl] * v
            return cc

        lax.fori_loop(0, CHUNK // 16, scale, 0)
        pltpu.sync_copy(rows, acc.at[dstv], add=True)
        return c

    lax.fori_loop(0, NCHUNK, chunk_body, 0)
    plsc.subcore_barrier()

    for k in range((NZB + NS - 1) // NS):
        b = k * NS + sid

        @pl.when(b < NZB)
        def _():
            base = pl.multiple_of(b * ZR, 8)
            pltpu.sync_copy(acc.at[pl.ds(base, ZR)], zbuf)
            pltpu.sync_copy(zbuf, out_hbm.at[cid, pl.ds(base, ZR)])


_spmm = pl.kernel(
    _spmm_body,
    out_type=jax.ShapeDtypeStruct((NC, N, D), jnp.float32),
    mesh=plsc.VectorSubcoreMesh(core_axis_name="c", subcore_axis_name="s"),
    scratch_types=[
        pltpu.VMEM_SHARED((N, D), jnp.float32),
        pltpu.VMEM((CHUNK,), jnp.int32),
        pltpu.VMEM((CHUNK,), jnp.int32),
        pltpu.VMEM((CHUNK,), jnp.float32),
        pltpu.VMEM((CHUNK, D), jnp.float32),
        pltpu.VMEM((ZR, D), jnp.float32),
        pltpu.SemaphoreType.DMA,
    ],
)


def _tc1_body(lf_ref, f_ref, w1_ref, b1_ref, inter_ref, part1_ref):
    lf = lf_ref[0] + lf_ref[1]
    f = f_ref[...]
    inter_ref[...] = lf * f
    part1_ref[...] = lax.dot_general(
        lf + f, w1_ref[...], (((1,), (1,)), ((), ())),
        preferred_element_type=jnp.float32) + b1_ref[...]


BR = 2000

_tc1 = pl.pallas_call(
    _tc1_body,
    grid=(N // BR,),
    in_specs=[
        pl.BlockSpec((NC, BR, D), lambda i: (0, i, 0)),
        pl.BlockSpec((BR, D), lambda i: (i, 0)),
        pl.BlockSpec((D, D), lambda i: (0, 0)),
        pl.BlockSpec((1, D), lambda i: (0, 0)),
    ],
    out_specs=[
        pl.BlockSpec((BR, D), lambda i: (i, 0)),
        pl.BlockSpec((BR, D), lambda i: (i, 0)),
    ],
    out_shape=[
        jax.ShapeDtypeStruct((N, D), jnp.float32),
        jax.ShapeDtypeStruct((N, D), jnp.float32),
    ],
)


def _tc2_body(part1_ref, p_ref, w2_ref, b2_ref, out_ref):
    p = p_ref[0] + p_ref[1]
    out_ref[...] = part1_ref[...] + lax.dot_general(
        p, w2_ref[...], (((1,), (1,)), ((), ())),
        preferred_element_type=jnp.float32) + b2_ref[...]


_tc2 = pl.pallas_call(
    _tc2_body,
    grid=(N // BR,),
    in_specs=[
        pl.BlockSpec((BR, D), lambda i: (i, 0)),
        pl.BlockSpec((NC, BR, D), lambda i: (0, i, 0)),
        pl.BlockSpec((D, D), lambda i: (0, 0)),
        pl.BlockSpec((1, D), lambda i: (0, 0)),
    ],
    out_specs=pl.BlockSpec((BR, D), lambda i: (i, 0)),
    out_shape=jax.ShapeDtypeStruct((N, D), jnp.float32),
)


def kernel(laplacian_indices, laplacian_values, features, W1, b1, W2, b2):
    dst = laplacian_indices[0]
    src = laplacian_indices[1]
    lf_parts = _spmm(src, dst, laplacian_values, features)
    inter, part1 = _tc1(lf_parts, features, W1, b1.reshape(1, D))
    p_parts = _spmm(src, dst, laplacian_values, inter)
    return _tc2(part1, p_parts, W2, b2.reshape(1, D))


# speedup vs baseline: 9.2764x; 2.2247x over previous
import jax
import jax.numpy as jnp
from jax import lax
from jax.experimental import pallas as pl
from jax.experimental.pallas import tpu as pltpu
from jax.experimental.pallas import tpu_sc as plsc

N = 10000
E = 320000
D = 128

NC = 2
NS = 16
NW = NC * NS
EPW = E // NW
CHUNK = 80
NCHUNK = EPW // CHUNK
NSB = 5
SBC = NCHUNK // NSB
ZR = 80
NZB = N // ZR


def _spmm_body(src_hbm, dst_hbm, vals_hbm, table_hbm, out_hbm,
               acc, srcv, dstv, valv, rows0, rows1, sem0, sem1):
    cid = lax.axis_index("c")
    sid = lax.axis_index("s")
    wid = cid * NS + sid
    rows = (rows0, rows1)
    sems = (sem0, sem1)

    zero16 = jnp.zeros((16,), jnp.float32)

    def zb(i, c):
        for j in range(8):
            rows1[i, pl.ds(j * 16, 16)] = zero16
        return c

    lax.fori_loop(0, ZR, zb, 0)

    for k in range((NZB + NS - 1) // NS):
        b = k * NS + sid

        @pl.when(b < NZB)
        def _():
            base = pl.multiple_of(b * ZR, 8)
            pltpu.sync_copy(rows1, acc.at[pl.ds(base, ZR)])

    plsc.subcore_barrier()

    def do_chunk(g, buf, prefetch):
        pltpu.make_async_copy(
            table_hbm.at[srcv.at[g]], rows[buf], sems[buf]).wait()
        if prefetch:
            pltpu.async_copy(
                table_hbm.at[srcv.at[g + 1]], rows[buf ^ 1], sems[buf ^ 1])

        def scale(grp, cc):
            vv = valv[g, pl.ds(grp * 16, 16)]
            rbase = grp * 16
            for r in range(16):
                v = vv[r]
                for j in range(8):
                    sl = pl.ds(j * 16, 16)
                    rows[buf][rbase + r, sl] = rows[buf][rbase + r, sl] * v
            return cc

        lax.fori_loop(0, CHUNK // 16, scale, 0)
        pltpu.sync_copy(rows[buf], acc.at[dstv.at[g]], add=True)

    def super_block(sb, c):
        pltpu.sync_copy(src_hbm.at[wid, sb], srcv)
        pltpu.sync_copy(dst_hbm.at[wid, sb], dstv)
        pltpu.sync_copy(vals_hbm.at[wid, sb], valv)
        pltpu.async_copy(table_hbm.at[srcv.at[0]], rows0, sem0)

        def pair(i, cc):
            g = i * 2
            do_chunk(g, 0, True)
            do_chunk(g + 1, 1, True)
            return cc

        lax.fori_loop(0, (SBC - 1) // 2, pair, 0)
        do_chunk(SBC - 1, 0, False)
        return c

    lax.fori_loop(0, NSB, super_block, 0)
    plsc.subcore_barrier()

    for k in range((NZB + NS - 1) // NS):
        b = k * NS + sid

        @pl.when(b < NZB)
        def _():
            base = pl.multiple_of(b * ZR, 8)
            pltpu.sync_copy(acc.at[pl.ds(base, ZR)], rows0)
            pltpu.sync_copy(rows0, out_hbm.at[cid, pl.ds(base, ZR)])


_spmm = pl.kernel(
    _spmm_body,
    out_type=jax.ShapeDtypeStruct((NC, N, D), jnp.float32),
    mesh=plsc.VectorSubcoreMesh(core_axis_name="c", subcore_axis_name="s"),
    scratch_types=[
        pltpu.VMEM_SHARED((N, D), jnp.float32),
        pltpu.VMEM((SBC, CHUNK), jnp.int32),
        pltpu.VMEM((SBC, CHUNK), jnp.int32),
        pltpu.VMEM((SBC, CHUNK), jnp.float32),
        pltpu.VMEM((CHUNK, D), jnp.float32),
        pltpu.VMEM((CHUNK, D), jnp.float32),
        pltpu.SemaphoreType.DMA,
        pltpu.SemaphoreType.DMA,
    ],
)


def _tc1_body(lf_ref, f_ref, w1_ref, b1_ref, inter_ref, part1_ref):
    lf = lf_ref[0] + lf_ref[1]
    f = f_ref[...]
    inter_ref[...] = lf * f
    part1_ref[...] = lax.dot_general(
        lf + f, w1_ref[...], (((1,), (1,)), ((), ())),
        preferred_element_type=jnp.float32) + b1_ref[...]


BR = 2000

_tc1 = pl.pallas_call(
    _tc1_body,
    grid=(N // BR,),
    in_specs=[
        pl.BlockSpec((NC, BR, D), lambda i: (0, i, 0)),
        pl.BlockSpec((BR, D), lambda i: (i, 0)),
        pl.BlockSpec((D, D), lambda i: (0, 0)),
        pl.BlockSpec((1, D), lambda i: (0, 0)),
    ],
    out_specs=[
        pl.BlockSpec((BR, D), lambda i: (i, 0)),
        pl.BlockSpec((BR, D), lambda i: (i, 0)),
    ],
    out_shape=[
        jax.ShapeDtypeStruct((N, D), jnp.float32),
        jax.ShapeDtypeStruct((N, D), jnp.float32),
    ],
)


def _tc2_body(part1_ref, p_ref, w2_ref, b2_ref, out_ref):
    p = p_ref[0] + p_ref[1]
    out_ref[...] = part1_ref[...] + lax.dot_general(
        p, w2_ref[...], (((1,), (1,)), ((), ())),
        preferred_element_type=jnp.float32) + b2_ref[...]


_tc2 = pl.pallas_call(
    _tc2_body,
    grid=(N // BR,),
    in_specs=[
        pl.BlockSpec((BR, D), lambda i: (i, 0)),
        pl.BlockSpec((NC, BR, D), lambda i: (0, i, 0)),
        pl.BlockSpec((D, D), lambda i: (0, 0)),
        pl.BlockSpec((1, D), lambda i: (0, 0)),
    ],
    out_specs=pl.BlockSpec((BR, D), lambda i: (i, 0)),
    out_shape=jax.ShapeDtypeStruct((N, D), jnp.float32),
)


def kernel(laplacian_indices, laplacian_values, features, W1, b1, W2, b2):
    dst = laplacian_indices[0].reshape(NW, NSB, SBC, CHUNK)
    src = laplacian_indices[1].reshape(NW, NSB, SBC, CHUNK)
    vals = laplacian_values.reshape(NW, NSB, SBC, CHUNK)
    lf_parts = _spmm(src, dst, vals, features)
    inter, part1 = _tc1(lf_parts, features, W1, b1.reshape(1, D))
    p_parts = _spmm(src, dst, vals, inter)
    return _tc2(part1, p_parts, W2, b2.reshape(1, D))
